# baseline (speedup 1.0000x reference)
import functools

import jax
import jax.numpy as jnp
from jax import lax
from jax.experimental import pallas as pl
from jax.experimental.pallas import tpu as pltpu
from jax.experimental.pallas import tpu_sc as plsc

N_NODES = 10000
N_PAD = 10240
N_EDGES = 320000
D = 128

NC, NS = 2, 16
NW = NC * NS
E_SHARD = 10240
E_PAD = NW * E_SHARD
CH = 80
NCHUNK = E_SHARD // CH
ROWS_T = N_PAD // NS
ZR = 80

_MESH = plsc.VectorSubcoreMesh(
    core_axis_name="c", subcore_axis_name="s", num_cores=NC, num_subcores=NS)


@functools.partial(
    pl.kernel,
    out_type=jax.ShapeDtypeStruct((NC * N_PAD,), jnp.float32),
    mesh=_MESH,
    scratch_types=[
        pltpu.VMEM((E_SHARD,), jnp.int32),
        pltpu.VMEM((CH,), jnp.float32),
        pltpu.VMEM((ROWS_T,), jnp.float32),
        pltpu.VMEM_SHARED((N_PAD,), jnp.float32),
        pltpu.SemaphoreType.DMA,
    ],
)
def _sc_degree(dst_hbm, out_hbm, dsti, onesv, zerov, hist, sem):
    c = lax.axis_index("c")
    s = lax.axis_index("s")
    shard_base = (s * NC + c) * E_SHARD
    row0 = s * ROWS_T
    pltpu.sync_copy(dst_hbm.at[pl.ds(shard_base, E_SHARD)], dsti)
    for k in range(CH // 16):
        onesv[pl.ds(k * 16, 16)] = jnp.ones((16,), jnp.float32)
    for k in range(ROWS_T // 16):
        zerov[pl.ds(k * 16, 16)] = jnp.zeros((16,), jnp.float32)
    pltpu.sync_copy(zerov, hist.at[pl.ds(row0, ROWS_T)])
    plsc.subcore_barrier()

    def body(j, carry):
        pltpu.sync_copy(onesv, hist.at[dsti.at[pl.ds(j * CH, CH)]], add=True)
        return carry

    lax.fori_loop(0, NCHUNK, body, 0)
    plsc.subcore_barrier()
    pltpu.sync_copy(hist.at[pl.ds(row0, ROWS_T)], zerov)
    pltpu.sync_copy(zerov, out_hbm.at[pl.ds(c * N_PAD + row0, ROWS_T)])


@functools.partial(
    pl.kernel,
    out_type=jax.ShapeDtypeStruct((NC * N_PAD, D), jnp.float32),
    mesh=_MESH,
    scratch_types=[
        pltpu.VMEM((E_SHARD,), jnp.int32),
        pltpu.VMEM((E_SHARD,), jnp.int32),
        pltpu.VMEM((CH, D), jnp.float32),
        pltpu.VMEM((CH, D), jnp.float32),
        pltpu.VMEM_SHARED((N_PAD, D), jnp.float32),
        pltpu.SemaphoreType.DMA,
        pltpu.SemaphoreType.DMA,
    ],
)
def _sc_aggregate(y_hbm, src_hbm, dst_hbm, zeros_hbm, out_hbm, srci, dsti,
                  rows0, rows1, acc, sem0, sem1):
    c = lax.axis_index("c")
    s = lax.axis_index("s")
    shard_base = (s * NC + c) * E_SHARD
    row0 = s * ROWS_T
    pltpu.sync_copy(src_hbm.at[pl.ds(shard_base, E_SHARD)], srci)
    pltpu.sync_copy(dst_hbm.at[pl.ds(shard_base, E_SHARD)], dsti)
    pltpu.sync_copy(zeros_hbm, rows1)
    for k in range(ROWS_T // ZR):
        pltpu.sync_copy(rows1, acc.at[pl.ds(row0 + k * ZR, ZR)])
    plsc.subcore_barrier()

    def body(j, carry):
        pltpu.async_copy(
            y_hbm.at[srci.at[pl.ds(j * CH, CH)]], rows0, sem0).wait()
        pltpu.sync_copy(rows0, acc.at[dsti.at[pl.ds(j * CH, CH)]], add=True)
        return carry

    with jax.named_scope("agg_loop"):
        lax.fori_loop(0, NCHUNK, body, 0)
    plsc.subcore_barrier()
    with jax.named_scope("agg_dump"):
        for k in range(ROWS_T // ZR):
            pltpu.sync_copy(acc.at[pl.ds(row0 + k * ZR, ZR)], rows0)
            pltpu.sync_copy(
                rows0, out_hbm.at[pl.ds(c * N_PAD + row0 + k * ZR, ZR)])


_BR = 400
_GRID = N_NODES // _BR


def _dis(degp_ref):
    seg = degp_ref[...]
    return lax.rsqrt(seg[:, 0] + seg[:, 1] + 1.0)


def _tc_y1_body(x_ref, w_ref, degp_ref, o_ref):
    dis = _dis(degp_ref)
    xw = jnp.dot(x_ref[...], w_ref[...], preferred_element_type=jnp.float32)
    o_ref[...] = xw * dis[:, None]


def _tc_mid_body(a0_ref, a1_ref, y_ref, degp_ref, b_ref, w_ref, o_ref):
    dis = _dis(degp_ref)
    h = jax.nn.relu(dis[:, None] * (a0_ref[...] + a1_ref[...] + y_ref[...])
                    + b_ref[...])
    o_ref[...] = jnp.dot(h, w_ref[...],
                         preferred_element_type=jnp.float32) * dis[:, None]


def _tc_out_body(a0_ref, a1_ref, y_ref, degp_ref, b_ref, o_ref):
    dis = _dis(degp_ref)
    o_ref[...] = jax.nn.relu(
        dis[:, None] * (a0_ref[...] + a1_ref[...] + y_ref[...]) + b_ref[...])


_ROWB = pl.BlockSpec((_BR, D), lambda i: (i, 0))
_DEGB = pl.BlockSpec((_BR, NC), lambda i: (i, 0))
_WB = pl.BlockSpec((D, D), lambda i: (0, 0))
_BB = pl.BlockSpec((1, D), lambda i: (0, 0))
_OUT = jax.ShapeDtypeStruct((N_NODES, D), jnp.float32)


def _tc_y1(x, w1, degp):
    return pl.pallas_call(
        _tc_y1_body, grid=(_GRID,),
        in_specs=[_ROWB, _WB, _DEGB], out_specs=_ROWB, out_shape=_OUT,
    )(x, w1, degp)


def _tc_mid(a0, a1, y, degp, b, w2):
    return pl.pallas_call(
        _tc_mid_body, grid=(_GRID,),
        in_specs=[_ROWB, _ROWB, _ROWB, _DEGB, _BB, _WB], out_specs=_ROWB,
        out_shape=_OUT,
    )(a0, a1, y, degp, b, w2)


def _tc_out(a0, a1, y, degp, b):
    return pl.pallas_call(
        _tc_out_body, grid=(_GRID,),
        in_specs=[_ROWB, _ROWB, _ROWB, _DEGB, _BB], out_specs=_ROWB,
        out_shape=_OUT,
    )(a0, a1, y, degp, b)


def kernel(x, edge_index, W1, b1, W2, b2):
    ei = edge_index.astype(jnp.int32)
    pad = E_PAD - N_EDGES
    src = jnp.concatenate([ei[0], jnp.zeros((pad,), jnp.int32)])
    pad_dst = N_NODES + (jnp.arange(pad, dtype=jnp.int32) % (N_PAD - N_NODES))
    dst = jnp.concatenate([ei[1], pad_dst])
    zeros128 = jnp.zeros((ZR, D), jnp.float32)
    b1r = b1.reshape(1, D)
    b2r = b2.reshape(1, D)

    degp = _sc_degree(dst).reshape(NC, N_PAD).T
    y1 = _tc_y1(x, W1, degp)

    aggp1 = _sc_aggregate(y1, src, dst, zeros128)
    y2 = _tc_mid(aggp1[:N_PAD], aggp1[N_PAD:], y1, degp, b1r, W2)

    aggp2 = _sc_aggregate(y2, src, dst, zeros128)
    return _tc_out(aggp2[:N_PAD], aggp2[N_PAD:], y2, degp, b2r)


# speedup vs baseline: 3.2370x; 3.2370x over previous
import functools

import jax
import jax.numpy as jnp
from jax import lax
from jax.experimental import pallas as pl
from jax.experimental.pallas import tpu as pltpu
from jax.experimental.pallas import tpu_sc as plsc

N_NODES = 10000
N_PAD = 10240
N_EDGES = 320000
D = 128

NC, NS = 2, 16
NW = NC * NS
E_SHARD = 10240
E_PAD = NW * E_SHARD
CH = 80
NCHUNK = E_SHARD // CH
ROWS_T = N_PAD // NS
ZR = 80

_MESH = plsc.VectorSubcoreMesh(
    core_axis_name="c", subcore_axis_name="s", num_cores=NC, num_subcores=NS)


@functools.partial(
    pl.kernel,
    out_type=jax.ShapeDtypeStruct((NC * N_PAD,), jnp.float32),
    mesh=_MESH,
    scratch_types=[
        pltpu.VMEM((E_SHARD,), jnp.int32),
        pltpu.VMEM((CH,), jnp.float32),
        pltpu.VMEM((ROWS_T,), jnp.float32),
        pltpu.VMEM_SHARED((N_PAD,), jnp.float32),
        pltpu.SemaphoreType.DMA,
    ],
)
def _sc_degree(dst_hbm, out_hbm, dsti, onesv, zerov, hist, sem):
    c = lax.axis_index("c")
    s = lax.axis_index("s")
    shard_base = (s * NC + c) * E_SHARD
    row0 = s * ROWS_T
    pltpu.sync_copy(dst_hbm.at[pl.ds(shard_base, E_SHARD)], dsti)
    for k in range(CH // 16):
        onesv[pl.ds(k * 16, 16)] = jnp.ones((16,), jnp.float32)
    for k in range(ROWS_T // 16):
        zerov[pl.ds(k * 16, 16)] = jnp.zeros((16,), jnp.float32)
    pltpu.sync_copy(zerov, hist.at[pl.ds(row0, ROWS_T)])
    plsc.subcore_barrier()

    def body(j, carry):
        pltpu.sync_copy(onesv, hist.at[dsti.at[pl.ds(j * CH, CH)]], add=True)
        return carry

    lax.fori_loop(0, NCHUNK, body, 0)
    plsc.subcore_barrier()
    pltpu.sync_copy(hist.at[pl.ds(row0, ROWS_T)], zerov)
    pltpu.sync_copy(zerov, out_hbm.at[pl.ds(c * N_PAD + row0, ROWS_T)])


@functools.partial(
    pl.kernel,
    out_type=jax.ShapeDtypeStruct((NC * N_PAD, D), jnp.float32),
    mesh=_MESH,
    scratch_types=[
        pltpu.VMEM((E_SHARD,), jnp.int32),
        pltpu.VMEM((E_SHARD,), jnp.int32),
        pltpu.VMEM((CH, D), jnp.float32),
        pltpu.VMEM((CH, D), jnp.float32),
        pltpu.VMEM_SHARED((N_PAD, D), jnp.float32),
        pltpu.SemaphoreType.DMA,
        pltpu.SemaphoreType.DMA,
    ],
)
def _sc_aggregate(y_hbm, src_hbm, dst_hbm, zeros_hbm, out_hbm, srci, dsti,
                  rows0, rows1, acc, sem0, sem1):
    c = lax.axis_index("c")
    s = lax.axis_index("s")
    shard_base = (s * NC + c) * E_SHARD
    row0 = s * ROWS_T
    pltpu.sync_copy(src_hbm.at[pl.ds(shard_base, E_SHARD)], srci)
    pltpu.sync_copy(dst_hbm.at[pl.ds(shard_base, E_SHARD)], dsti)
    pltpu.sync_copy(zeros_hbm, rows1)
    for k in range(ROWS_T // ZR):
        pltpu.sync_copy(rows1, acc.at[pl.ds(row0 + k * ZR, ZR)])
    plsc.subcore_barrier()

    def gather(j, buf, sem):
        pltpu.async_copy(y_hbm.at[srci.at[pl.ds(j * CH, CH)]], buf, sem)

    def gather_wait(j, buf, sem):
        pltpu.make_async_copy(
            y_hbm.at[srci.at[pl.ds(j * CH, CH)]], buf, sem).wait()

    def scatter(j, buf):
        pltpu.sync_copy(buf, acc.at[dsti.at[pl.ds(j * CH, CH)]], add=True)

    gather(0, rows0, sem0)
    gather(1, rows1, sem1)

    def pair(jj, carry):
        j0 = jj * 2
        gather_wait(j0, rows0, sem0)
        scatter(j0, rows0)

        @pl.when(j0 + 2 < NCHUNK)
        def _():
            gather(j0 + 2, rows0, sem0)

        j1 = j0 + 1
        gather_wait(j1, rows1, sem1)
        scatter(j1, rows1)

        @pl.when(j1 + 2 < NCHUNK)
        def _():
            gather(j1 + 2, rows1, sem1)

        return carry

    lax.fori_loop(0, NCHUNK // 2, pair, 0)
    plsc.subcore_barrier()
    with jax.named_scope("agg_dump"):
        for k in range(ROWS_T // ZR):
            pltpu.sync_copy(acc.at[pl.ds(row0 + k * ZR, ZR)], rows0)
            pltpu.sync_copy(
                rows0, out_hbm.at[pl.ds(c * N_PAD + row0 + k * ZR, ZR)])


_BR = 400
_GRID = N_NODES // _BR


def _dis(degp_ref):
    seg = degp_ref[...]
    return lax.rsqrt(seg[:, 0] + seg[:, 1] + 1.0)


def _tc_y1_body(x_ref, w_ref, degp_ref, o_ref):
    dis = _dis(degp_ref)
    xw = jnp.dot(x_ref[...], w_ref[...], preferred_element_type=jnp.float32)
    o_ref[...] = xw * dis[:, None]


def _tc_mid_body(a0_ref, a1_ref, y_ref, degp_ref, b_ref, w_ref, o_ref):
    dis = _dis(degp_ref)
    h = jax.nn.relu(dis[:, None] * (a0_ref[...] + a1_ref[...] + y_ref[...])
                    + b_ref[...])
    o_ref[...] = jnp.dot(h, w_ref[...],
                         preferred_element_type=jnp.float32) * dis[:, None]


def _tc_out_body(a0_ref, a1_ref, y_ref, degp_ref, b_ref, o_ref):
    dis = _dis(degp_ref)
    o_ref[...] = jax.nn.relu(
        dis[:, None] * (a0_ref[...] + a1_ref[...] + y_ref[...]) + b_ref[...])


_ROWB = pl.BlockSpec((_BR, D), lambda i: (i, 0))
_DEGB = pl.BlockSpec((_BR, NC), lambda i: (i, 0))
_WB = pl.BlockSpec((D, D), lambda i: (0, 0))
_BB = pl.BlockSpec((1, D), lambda i: (0, 0))
_OUT = jax.ShapeDtypeStruct((N_NODES, D), jnp.float32)


def _tc_y1(x, w1, degp):
    return pl.pallas_call(
        _tc_y1_body, grid=(_GRID,),
        in_specs=[_ROWB, _WB, _DEGB], out_specs=_ROWB, out_shape=_OUT,
    )(x, w1, degp)


def _tc_mid(a0, a1, y, degp, b, w2):
    return pl.pallas_call(
        _tc_mid_body, grid=(_GRID,),
        in_specs=[_ROWB, _ROWB, _ROWB, _DEGB, _BB, _WB], out_specs=_ROWB,
        out_shape=_OUT,
    )(a0, a1, y, degp, b, w2)


def _tc_out(a0, a1, y, degp, b):
    return pl.pallas_call(
        _tc_out_body, grid=(_GRID,),
        in_specs=[_ROWB, _ROWB, _ROWB, _DEGB, _BB], out_specs=_ROWB,
        out_shape=_OUT,
    )(a0, a1, y, degp, b)


def kernel(x, edge_index, W1, b1, W2, b2):
    ei = edge_index.astype(jnp.int32)
    pad = E_PAD - N_EDGES
    idx = jnp.arange(pad, dtype=jnp.int32)
    src = jnp.concatenate([ei[0], idx % N_NODES])
    dst = jnp.concatenate([ei[1], N_NODES + idx % (N_PAD - N_NODES)])
    zeros128 = jnp.zeros((ZR, D), jnp.float32)
    b1r = b1.reshape(1, D)
    b2r = b2.reshape(1, D)

    degp = _sc_degree(dst).reshape(NC, N_PAD).T
    y1 = _tc_y1(x, W1, degp)

    aggp1 = _sc_aggregate(y1, src, dst, zeros128)
    y2 = _tc_mid(aggp1[:N_PAD], aggp1[N_PAD:], y1, degp, b1r, W2)

    aggp2 = _sc_aggregate(y2, src, dst, zeros128)
    return _tc_out(aggp2[:N_PAD], aggp2[N_PAD:], y2, degp, b2r)
